# baseline (speedup 1.0000x reference)
import functools

import jax
import jax.numpy as jnp
from jax import lax
from jax.experimental import pallas as pl
from jax.experimental.pallas import tpu as pltpu
from jax.experimental.pallas import tpu_sc as plsc

EMB = 256
LANES = 16
SCH = 96
TCH = 64
PAIR = SCH + TCH
NUM_TILES = 32


def _prep_body(w0_ref, w1_ref, w2_ref, a0_ref, a1_ref, a2_ref, t_ref, c_ref):
    for a0 in range(w0_ref.shape[0]):
        for a1 in range(w1_ref.shape[0]):
            for a2 in range(w2_ref.shape[0]):
                c = (a0 * w1_ref.shape[0] + a1) * w2_ref.shape[0] + a2
                t_ref[0, c, :] = w0_ref[a0, :] + w1_ref[a1, :] + w2_ref[a2, :]
    for k in range(1, NUM_TILES):
        t_ref[k, :, :] = t_ref[0, :, :]
    n1 = w1_ref.shape[0]
    n2 = w2_ref.shape[0]
    c_ref[...] = (a0_ref[...] * n1 + a1_ref[...]) * n2 + a2_ref[...]


def _make_sc_kernel(num_edges, ncombo):
    npairs = num_edges // PAIR
    base_cnt = npairs // NUM_TILES
    rem = npairs % NUM_TILES
    iters = base_cnt + (1 if rem else 0)
    strip = iters * PAIR
    mesh = plsc.VectorSubcoreMesh(core_axis_name="c", subcore_axis_name="s")

    @functools.partial(
        pl.kernel,
        mesh=mesh,
        out_type=jax.ShapeDtypeStruct((num_edges, EMB), jnp.float32),
        scratch_types=[
            pltpu.VMEM((strip,), jnp.int32),
            pltpu.VMEM((3 * SCH, EMB), jnp.float32),
            pltpu.VMEM((2 * TCH, EMB), jnp.float32),
            pltpu.VMEM((ncombo * EMB,), jnp.float32),
            pltpu.SemaphoreType.DMA,
            pltpu.SemaphoreType.DMA,
            pltpu.SemaphoreType.DMA,
            pltpu.SemaphoreType.DMA,
        ],
    )
    def sc_build(t_rep_hbm, t_flat_hbm, c_hbm, out_hbm, idx_v, sbuf, tbuf,
                 t_v, g_sem0, g_sem1, ws_sem, wt_sem):
        cid = lax.axis_index("c")
        sid = lax.axis_index("s")
        w = sid * 2 + cid

        start = w * base_cnt + jnp.minimum(w, rem)
        count = jnp.where(w < rem, base_cnt + 1, base_cnt)

        pltpu.sync_copy(t_flat_hbm, t_v)

        if rem:
            @pl.when(w < rem)
            def _():
                pltpu.sync_copy(
                    c_hbm.at[pl.ds(start * PAIR, (base_cnt + 1) * PAIR)],
                    idx_v)

            @pl.when(w >= rem)
            def _():
                pltpu.sync_copy(
                    c_hbm.at[pl.ds(start * PAIR, base_cnt * PAIR)],
                    idx_v.at[pl.ds(0, base_cnt * PAIR)])
        else:
            pltpu.sync_copy(c_hbm.at[pl.ds(start * PAIR, base_cnt * PAIR)],
                            idx_v)

        off = w * ncombo

        @plsc.parallel_loop(0, strip // LANES, unroll=4)
        def rebase(k):
            idx_v[pl.ds(k * LANES, LANES)] = (
                idx_v[pl.ds(k * LANES, LANES)] + off)

        def gather_start(p, rb, sem):
            pltpu.async_copy(
                t_rep_hbm.at[idx_v.at[pl.ds(p * PAIR, SCH)]],
                sbuf.at[pl.ds(rb * SCH, SCH), :], sem)

        def gather_wait(rb, sem):
            pltpu.make_async_copy(t_rep_hbm.at[idx_v.at[pl.ds(0, SCH)]],
                                  sbuf.at[pl.ds(rb * SCH, SCH), :],
                                  sem).wait()

        def swrite_start(p, rb):
            pltpu.async_copy(
                sbuf.at[pl.ds(rb * SCH, SCH), :],
                out_hbm.at[pl.ds((start + p) * PAIR, SCH), :], ws_sem)

        def swrite_wait():
            pltpu.make_async_copy(sbuf.at[pl.ds(0, SCH), :],
                                  out_hbm.at[pl.ds(0, SCH), :],
                                  ws_sem).wait()

        def build_chunk(p, tb):
            @plsc.parallel_loop(0, TCH // LANES, unroll=1)
            def grp(g):
                cv = (idx_v[pl.ds(p * PAIR + SCH + g * LANES, LANES)]
                      - off) * EMB
                rbase = tb * TCH + g * LANES
                for k in range(LANES):
                    c = cv[k]
                    vals = [t_v[pl.ds(c + j * LANES, LANES)]
                            for j in range(EMB // LANES)]
                    for j, v in enumerate(vals):
                        tbuf[rbase + k, pl.ds(j * LANES, LANES)] = v

        def twrite_start(p, tb):
            pltpu.async_copy(
                tbuf.at[pl.ds(tb * TCH, TCH), :],
                out_hbm.at[pl.ds((start + p) * PAIR + SCH, TCH), :], wt_sem)

        def twrite_wait():
            pltpu.make_async_copy(tbuf.at[pl.ds(0, TCH), :],
                                  out_hbm.at[pl.ds(0, TCH), :],
                                  wt_sem).wait()

        gather_start(0, 0, g_sem0)
        gather_start(1, 1, g_sem1)

        def body(j, carry):
            rb = lax.rem(j, 3)
            tb = lax.rem(j, 2)
            even = lax.rem(j, 2) == 0

            @pl.when(j < count)
            def _():
                @pl.when(even)
                def _():
                    gather_wait(rb, g_sem0)

                @pl.when(jnp.logical_not(even))
                def _():
                    gather_wait(rb, g_sem1)

                @pl.when(j >= 1)
                def _():
                    swrite_wait()

                @pl.when(jnp.logical_and(even, j + 2 < count))
                def _():
                    gather_start(j + 2, lax.rem(j + 2, 3), g_sem0)

                @pl.when(jnp.logical_and(jnp.logical_not(even),
                                         j + 2 < count))
                def _():
                    gather_start(j + 2, lax.rem(j + 2, 3), g_sem1)

                swrite_start(j, rb)

                @pl.when(j >= 2)
                def _():
                    twrite_wait()

                build_chunk(j, tb)
                twrite_start(j, tb)

            return carry

        lax.fori_loop(0, iters, body, 0)
        swrite_wait()
        twrite_wait()
        twrite_wait()

    return sc_build


def kernel(edge_attr, W0, W1, W2):
    num_edges = edge_attr.shape[0]
    attr = edge_attr.astype(jnp.int32)
    rows = num_edges // PAIR
    a0 = attr[:, 0].reshape(rows, PAIR)
    a1 = attr[:, 1].reshape(rows, PAIR)
    a2 = attr[:, 2].reshape(rows, PAIR)

    ncombo = W0.shape[0] * W1.shape[0] * W2.shape[0]
    t_rep, c2d = pl.pallas_call(
        _prep_body,
        out_shape=(
            jax.ShapeDtypeStruct((NUM_TILES, ncombo, EMB), jnp.float32),
            jax.ShapeDtypeStruct((rows, PAIR), jnp.int32),
        ),
    )(W0, W1, W2, a0, a1, a2)

    return _make_sc_kernel(num_edges, ncombo)(
        t_rep.reshape(NUM_TILES * ncombo, EMB),
        t_rep[0].reshape(ncombo * EMB),
        c2d.reshape(num_edges))


# speedup vs baseline: 1.0474x; 1.0474x over previous
import functools

import jax
import jax.numpy as jnp
from jax import lax
from jax.experimental import pallas as pl
from jax.experimental.pallas import tpu as pltpu
from jax.experimental.pallas import tpu_sc as plsc

EMB = 256
LANES = 16
SCH = 80
TCH = 80
PAIR = SCH + TCH
NUM_TILES = 32


def _prep_body(w0_ref, w1_ref, w2_ref, a0_ref, a1_ref, a2_ref, t_ref, c_ref):
    for a0 in range(w0_ref.shape[0]):
        for a1 in range(w1_ref.shape[0]):
            for a2 in range(w2_ref.shape[0]):
                c = (a0 * w1_ref.shape[0] + a1) * w2_ref.shape[0] + a2
                t_ref[0, c, :] = w0_ref[a0, :] + w1_ref[a1, :] + w2_ref[a2, :]
    for k in range(1, NUM_TILES):
        t_ref[k, :, :] = t_ref[0, :, :]
    n1 = w1_ref.shape[0]
    n2 = w2_ref.shape[0]
    c_ref[...] = (a0_ref[...] * n1 + a1_ref[...]) * n2 + a2_ref[...]


def _make_sc_kernel(num_edges, ncombo):
    npairs = num_edges // PAIR
    base_cnt = npairs // NUM_TILES
    rem = npairs % NUM_TILES
    iters = base_cnt + (1 if rem else 0)
    strip = iters * PAIR
    mesh = plsc.VectorSubcoreMesh(core_axis_name="c", subcore_axis_name="s")

    @functools.partial(
        pl.kernel,
        mesh=mesh,
        out_type=jax.ShapeDtypeStruct((num_edges, EMB), jnp.float32),
        scratch_types=[
            pltpu.VMEM((strip,), jnp.int32),
            pltpu.VMEM((2 * SCH, EMB), jnp.float32),
            pltpu.VMEM((2 * TCH, EMB), jnp.float32),
            pltpu.VMEM((ncombo * EMB,), jnp.float32),
            pltpu.SemaphoreType.DMA,
            pltpu.SemaphoreType.DMA,
            pltpu.SemaphoreType.DMA,
            pltpu.SemaphoreType.DMA,
        ],
    )
    def sc_build(t_rep_hbm, t_flat_hbm, c_hbm, out_hbm, idx_v, sbuf, tbuf,
                 t_v, g_sem0, g_sem1, ws_sem, wt_sem):
        cid = lax.axis_index("c")
        sid = lax.axis_index("s")
        w = sid * 2 + cid

        start = w * base_cnt + jnp.minimum(w, rem)
        count = jnp.where(w < rem, base_cnt + 1, base_cnt)

        pltpu.sync_copy(t_flat_hbm, t_v)

        if rem:
            @pl.when(w < rem)
            def _():
                pltpu.sync_copy(
                    c_hbm.at[pl.ds(start * PAIR, (base_cnt + 1) * PAIR)],
                    idx_v)

            @pl.when(w >= rem)
            def _():
                pltpu.sync_copy(
                    c_hbm.at[pl.ds(start * PAIR, base_cnt * PAIR)],
                    idx_v.at[pl.ds(0, base_cnt * PAIR)])
        else:
            pltpu.sync_copy(c_hbm.at[pl.ds(start * PAIR, base_cnt * PAIR)],
                            idx_v)

        off = w * ncombo

        @plsc.parallel_loop(0, strip // LANES, unroll=4)
        def rebase(k):
            idx_v[pl.ds(k * LANES, LANES)] = (
                idx_v[pl.ds(k * LANES, LANES)] + off)

        def gather_start(p, rb, sem):
            pltpu.async_copy(
                t_rep_hbm.at[idx_v.at[pl.ds(p * PAIR, SCH)]],
                sbuf.at[pl.ds(rb * SCH, SCH), :], sem)

        def gather_wait(rb, sem):
            pltpu.make_async_copy(t_rep_hbm.at[idx_v.at[pl.ds(0, SCH)]],
                                  sbuf.at[pl.ds(rb * SCH, SCH), :],
                                  sem).wait()

        def swrite_start(p, rb):
            pltpu.async_copy(
                sbuf.at[pl.ds(rb * SCH, SCH), :],
                out_hbm.at[pl.ds((start + p) * PAIR, SCH), :], ws_sem)

        def swrite_wait():
            pltpu.make_async_copy(sbuf.at[pl.ds(0, SCH), :],
                                  out_hbm.at[pl.ds(0, SCH), :],
                                  ws_sem).wait()

        def build_chunk(p, tb):
            @plsc.parallel_loop(0, TCH // LANES, unroll=1)
            def grp(g):
                cv = (idx_v[pl.ds(p * PAIR + SCH + g * LANES, LANES)]
                      - off) * EMB
                rbase = tb * TCH + g * LANES
                for k in range(LANES):
                    c = cv[k]
                    vals = [t_v[pl.ds(c + j * LANES, LANES)]
                            for j in range(EMB // LANES)]
                    for j, v in enumerate(vals):
                        tbuf[rbase + k, pl.ds(j * LANES, LANES)] = v

        def twrite_start(p, tb):
            pltpu.async_copy(
                tbuf.at[pl.ds(tb * TCH, TCH), :],
                out_hbm.at[pl.ds((start + p) * PAIR + SCH, TCH), :], wt_sem)

        def twrite_wait():
            pltpu.make_async_copy(tbuf.at[pl.ds(0, TCH), :],
                                  out_hbm.at[pl.ds(0, TCH), :],
                                  wt_sem).wait()

        gather_start(0, 0, g_sem0)

        def body(j, carry):
            rb = lax.rem(j, 2)
            tb = lax.rem(j, 2)
            even = rb == 0

            @pl.when(j < count)
            def _():
                @pl.when(j >= 1)
                def _():
                    swrite_wait()

                @pl.when(jnp.logical_and(even, j + 1 < count))
                def _():
                    gather_start(j + 1, 1 - rb, g_sem1)

                @pl.when(jnp.logical_and(jnp.logical_not(even),
                                         j + 1 < count))
                def _():
                    gather_start(j + 1, 1 - rb, g_sem0)

                @pl.when(j >= 2)
                def _():
                    twrite_wait()

                build_chunk(j, tb)
                twrite_start(j, tb)

                @pl.when(even)
                def _():
                    gather_wait(rb, g_sem0)

                @pl.when(jnp.logical_not(even))
                def _():
                    gather_wait(rb, g_sem1)

                swrite_start(j, rb)

            return carry

        lax.fori_loop(0, iters, body, 0)
        swrite_wait()
        twrite_wait()
        twrite_wait()

    return sc_build


def kernel(edge_attr, W0, W1, W2):
    num_edges = edge_attr.shape[0]
    attr = edge_attr.astype(jnp.int32)
    rows = num_edges // PAIR
    a0 = attr[:, 0].reshape(rows, PAIR)
    a1 = attr[:, 1].reshape(rows, PAIR)
    a2 = attr[:, 2].reshape(rows, PAIR)

    ncombo = W0.shape[0] * W1.shape[0] * W2.shape[0]
    t_rep, c2d = pl.pallas_call(
        _prep_body,
        out_shape=(
            jax.ShapeDtypeStruct((NUM_TILES, ncombo, EMB), jnp.float32),
            jax.ShapeDtypeStruct((rows, PAIR), jnp.int32),
        ),
    )(W0, W1, W2, a0, a1, a2)

    return _make_sc_kernel(num_edges, ncombo)(
        t_rep.reshape(NUM_TILES * ncombo, EMB),
        t_rep[0].reshape(ncombo * EMB),
        c2d.reshape(num_edges))
